# Initial kernel scaffold; baseline (speedup 1.0000x reference)
import jax
import jax.numpy as jnp
from jax.experimental import pallas as pl


def kernel(sequence, transitions, Wi, Wf, Wo, Wu, Uil, Uir, Ufl, Ufr, Uol, Uor, Uul, Uur, bi, bf, bo, bu):
    raise NotImplementedError("write your pallas kernel here")


# speedup vs baseline: 11.6831x; 11.6831x over previous
import functools

import jax
import jax.numpy as jnp
from jax import lax
from jax.experimental import pallas as pl
from jax.experimental.pallas import tpu as pltpu
from jax.experimental.pallas import tpu_sc as plsc

B = 64
L = 255
T = 255
H = 256
D = 2 * H
NW = 4
TB = T * 16
CHUNK = 120


def _sc_body(trans_hbm, seq_hbm, sp1_hbm, sp2_hbm, buf_hbm,
             bq_v, sp1_v, sp2_v, gidx_v, trans_v, rows_v, sem):
    wid = lax.axis_index("s") * 2 + lax.axis_index("c")

    @pl.when(wid < NW)
    def _():
        pltpu.sync_copy(trans_hbm.at[wid], trans_v)
        lane = lax.iota(jnp.int32, 16)
        bvec = wid.astype(jnp.int32) * 16 + lane
        zeros = jnp.zeros((16,), jnp.int32)

        def step(t, carry):
            blen, bptr = carry
            i1 = jnp.clip(blen - 1, 0, T - 1)
            i2 = jnp.clip(blen - 2, 0, T - 1)
            q1 = plsc.load_gather(bq_v, [i1 * 16 + lane])
            q2 = plsc.load_gather(bq_v, [i2 * 16 + lane])
            sp1 = jnp.where(blen >= 1, q1, zeros)
            sp2 = jnp.where(blen >= 2, q2, zeros)
            m = trans_v[pl.ds(t * 16, 16)]
            sp1_v[pl.ds(t * 16, 16)] = sp1
            sp2_v[pl.ds(t * 16, 16)] = sp2
            gidx_v[pl.ds(t * 16, 16)] = bvec * L + jnp.clip(bptr, 0, L - 1)
            nl = jnp.maximum(jnp.where(m == 1, blen - 2, blen), zeros)
            plsc.store_scatter(bq_v, [jnp.clip(nl, 0, T - 1) * 16 + lane],
                               jnp.full((16,), 1, jnp.int32) * (t + 1))
            return (nl + 1, bptr + (1 - m))

        lax.fori_loop(0, T, step, (zeros, zeros))
        pltpu.sync_copy(sp1_v, sp1_hbm.at[wid])
        pltpu.sync_copy(sp2_v, sp2_hbm.at[wid])

        def chunk(c, _):
            pltpu.async_copy(
                seq_hbm.at[gidx_v.at[pl.ds(c * CHUNK, CHUNK)]], rows_v, sem,
            ).wait()
            pltpu.sync_copy(rows_v, buf_hbm.at[wid, pl.ds(c * CHUNK, CHUNK), :])
            return 0

        lax.fori_loop(0, TB // CHUNK, chunk, 0)


@functools.lru_cache(maxsize=1)
def _make_sc_call():
    return functools.partial(
        pl.kernel,
        out_type=(
            jax.ShapeDtypeStruct((NW, TB), jnp.int32),
            jax.ShapeDtypeStruct((NW, TB), jnp.int32),
            jax.ShapeDtypeStruct((NW, TB, D), jnp.float32),
        ),
        mesh=plsc.VectorSubcoreMesh(core_axis_name="c", subcore_axis_name="s"),
        compiler_params=pltpu.CompilerParams(needs_layout_passes=False),
        scratch_types=[
            pltpu.VMEM((TB,), jnp.int32),
            pltpu.VMEM((TB,), jnp.int32),
            pltpu.VMEM((TB,), jnp.int32),
            pltpu.VMEM((TB,), jnp.int32),
            pltpu.VMEM((TB,), jnp.int32),
            pltpu.VMEM((CHUNK, D), jnp.float32),
            pltpu.SemaphoreType.DMA,
        ],
    )(_sc_body)


def _tc_body(s1_ref, s2_ref, buf_ref, mask_ref, ul_ref, ur_ref, bias_ref,
             out_ref, stack_ref, right_ref, left_ref):
    k = pl.program_id(0)

    @pl.when(k == 0)
    def _():
        stack_ref[0] = jnp.zeros((B, D), jnp.float32)

    def gather_b(b, _):
        s1 = s1_ref[k * B + b]
        s2 = s2_ref[k * B + b]
        right_ref[pl.ds(b, 1), :] = stack_ref[s1, pl.ds(b, 1), :]
        left_ref[pl.ds(b, 1), :] = stack_ref[s2, pl.ds(b, 1), :]
        return 0

    lax.fori_loop(0, B, gather_b, 0, unroll=2)

    hl = right_ref[:, :H]
    cl = right_ref[:, H:]
    hr = left_ref[:, :H]
    cr = left_ref[:, H:]
    acc = (jnp.dot(hl, ul_ref[:, :], preferred_element_type=jnp.float32)
           + jnp.dot(hr, ur_ref[:, :], preferred_element_type=jnp.float32)
           + bias_ref[:, :])
    i_g = jax.nn.sigmoid(acc[:, 0:H])
    o_g = jax.nn.sigmoid(acc[:, H:2 * H])
    f_g = jax.nn.sigmoid(acc[:, 2 * H:3 * H])
    u_g = jnp.tanh(acc[:, 3 * H:])
    c_j = i_g * u_g + f_g * (cl + cr)
    h_j = o_g * jnp.tanh(c_j)
    hc = jnp.concatenate([h_j, c_j], axis=1)

    m = mask_ref[0]
    buf = buf_ref[:, 0, :, :].reshape(B, D)
    row = m * hc + (1.0 - m) * buf
    stack_ref[k + 1] = row

    @pl.when(k == T - 1)
    def _():
        out_ref[:, :] = row[:, :H]


_tc_grid_spec = pltpu.PrefetchScalarGridSpec(
    num_scalar_prefetch=2,
    grid=(T,),
    in_specs=[
        pl.BlockSpec((NW, 1, 16, D), lambda i, s1, s2: (0, i, 0, 0)),
        pl.BlockSpec((1, B, 1), lambda i, s1, s2: (i, 0, 0)),
        pl.BlockSpec((H, 4 * H), lambda i, s1, s2: (0, 0)),
        pl.BlockSpec((H, 4 * H), lambda i, s1, s2: (0, 0)),
        pl.BlockSpec((1, 4 * H), lambda i, s1, s2: (0, 0)),
    ],
    out_specs=pl.BlockSpec((B, H), lambda i, s1, s2: (0, 0)),
    scratch_shapes=[
        pltpu.VMEM((T + 1, B, D), jnp.float32),
        pltpu.VMEM((B, D), jnp.float32),
        pltpu.VMEM((B, D), jnp.float32),
    ],
)


def kernel(sequence, transitions, Wi, Wf, Wo, Wu, Uil, Uir, Ufl, Ufr,
           Uol, Uor, Uul, Uur, bi, bf, bo, bu):
    del Wi, Wf, Wo, Wu
    trans32 = transitions.astype(jnp.int32)
    trans_prep = trans32.reshape(NW, 16, T).transpose(0, 2, 1).reshape(NW, TB)
    seqflat = sequence.reshape(B * L, D)

    sp1w, sp2w, bufw = _make_sc_call()(trans_prep, seqflat)

    sp1flat = sp1w.reshape(NW, T, 16).transpose(1, 0, 2).reshape(T * B)
    sp2flat = sp2w.reshape(NW, T, 16).transpose(1, 0, 2).reshape(T * B)
    buf4 = bufw.reshape(NW, T, 16, D)
    maskcol = trans32.T.astype(jnp.float32).reshape(T, B, 1)

    ULcat = jnp.concatenate([Uil.T, Uol.T, Ufl.T, Uul.T], axis=1)
    URcat = jnp.concatenate([Uir.T, Uor.T, Ufr.T, Uur.T], axis=1)
    bcat = jnp.concatenate([bi, bo, bf, bu]).reshape(1, 4 * H)

    return pl.pallas_call(
        _tc_body,
        grid_spec=_tc_grid_spec,
        out_shape=jax.ShapeDtypeStruct((B, H), jnp.float32),
        compiler_params=pltpu.CompilerParams(
            dimension_semantics=("arbitrary",)),
    )(sp1flat, sp2flat, buf4, maskcol, ULcat, URcat, bcat)


# speedup vs baseline: 13.9411x; 1.1933x over previous
import functools

import jax
import jax.numpy as jnp
from jax import lax
from jax.experimental import pallas as pl
from jax.experimental.pallas import tpu as pltpu
from jax.experimental.pallas import tpu_sc as plsc

B = 64
L = 255
T = 255
H = 256
D = 2 * H
NW = 4
TB = T * 16
CHUNK = 80
NCHUNK = TB // CHUNK
NSLICE = 8
CPT = 7


def _sc_body(trans_hbm, seq_hbm, sp1_hbm, sp2_hbm, buf_hbm,
             bq_v, sp1_v, sp2_v, gidx_v, trans_v, rows_a, rows_b,
             sga, sgb, ssa, ssb):
    wid = lax.axis_index("s") * 2 + lax.axis_index("c")
    grp = wid % NW
    slc = wid // NW

    pltpu.sync_copy(trans_hbm.at[grp], trans_v)
    lane = lax.iota(jnp.int32, 16)
    bvec = grp.astype(jnp.int32) * 16 + lane
    zeros = jnp.zeros((16,), jnp.int32)

    def step(t, carry):
        blen, bptr = carry
        i1 = jnp.clip(blen - 1, 0, T - 1)
        i2 = jnp.clip(blen - 2, 0, T - 1)
        q1 = plsc.load_gather(bq_v, [i1 * 16 + lane])
        q2 = plsc.load_gather(bq_v, [i2 * 16 + lane])
        sp1 = jnp.where(blen >= 1, q1, zeros)
        sp2 = jnp.where(blen >= 2, q2, zeros)
        m = trans_v[pl.ds(t * 16, 16)]
        sp1_v[pl.ds(t * 16, 16)] = sp1
        sp2_v[pl.ds(t * 16, 16)] = sp2
        gidx_v[pl.ds(t * 16, 16)] = bvec * L + jnp.clip(bptr, 0, L - 1)
        nl = jnp.maximum(jnp.where(m == 1, blen - 2, blen), zeros)
        plsc.store_scatter(bq_v, [jnp.clip(nl, 0, T - 1) * 16 + lane],
                           jnp.full((16,), 1, jnp.int32) * (t + 1))
        return (nl + 1, bptr + (1 - m))

    lax.fori_loop(0, T, step, (zeros, zeros))

    @pl.when(slc == 0)
    def _():
        pltpu.sync_copy(sp1_v, sp1_hbm.at[grp])
        pltpu.sync_copy(sp2_v, sp2_hbm.at[grp])

    bufs = (rows_a, rows_b)
    gsem = (sga, sgb)
    ssem = (ssa, ssb)
    pend = [None, None]
    for j in range(CPT):
        p = j % 2
        c = slc + NSLICE * j
        c = jnp.where(c < NCHUNK, c, c - NCHUNK)
        off = c * CHUNK
        if pend[p] is not None:
            pend[p].wait()
        pltpu.async_copy(
            seq_hbm.at[gidx_v.at[pl.ds(off, CHUNK)]], bufs[p], gsem[p],
        ).wait()
        pend[p] = pltpu.async_copy(
            bufs[p], buf_hbm.at[grp, pl.ds(off, CHUNK), :], ssem[p])
    pend[0].wait()
    pend[1].wait()


@functools.lru_cache(maxsize=1)
def _make_sc_call():
    return functools.partial(
        pl.kernel,
        out_type=(
            jax.ShapeDtypeStruct((NW, TB), jnp.int32),
            jax.ShapeDtypeStruct((NW, TB), jnp.int32),
            jax.ShapeDtypeStruct((NW, TB, D), jnp.float32),
        ),
        mesh=plsc.VectorSubcoreMesh(core_axis_name="c", subcore_axis_name="s"),
        compiler_params=pltpu.CompilerParams(needs_layout_passes=False),
        scratch_types=[
            pltpu.VMEM((TB,), jnp.int32),
            pltpu.VMEM((TB,), jnp.int32),
            pltpu.VMEM((TB,), jnp.int32),
            pltpu.VMEM((TB,), jnp.int32),
            pltpu.VMEM((TB,), jnp.int32),
            pltpu.VMEM((CHUNK, D), jnp.float32),
            pltpu.VMEM((CHUNK, D), jnp.float32),
            pltpu.SemaphoreType.DMA,
            pltpu.SemaphoreType.DMA,
            pltpu.SemaphoreType.DMA,
            pltpu.SemaphoreType.DMA,
        ],
    )(_sc_body)


def _tc_body(s1_ref, s2_ref, buf_ref, mask_ref, ul_ref, ur_ref, bias_ref,
             out_ref, stack_ref, right_ref, left_ref):
    k = pl.program_id(0)

    @pl.when(k == 0)
    def _():
        stack_ref[0] = jnp.zeros((B, D), jnp.float32)

    def gather_b(b, _):
        flat = (b // 16) * TB + k * 16 + (b % 16)
        s1 = s1_ref[flat]
        s2 = s2_ref[flat]
        right_ref[pl.ds(b, 1), :] = stack_ref[s1, pl.ds(b, 1), :]
        left_ref[pl.ds(b, 1), :] = stack_ref[s2, pl.ds(b, 1), :]
        return 0

    lax.fori_loop(0, B, gather_b, 0, unroll=2)

    hl = right_ref[:, :H]
    cl = right_ref[:, H:]
    hr = left_ref[:, :H]
    cr = left_ref[:, H:]
    acc = (jnp.dot(hl, ul_ref[:, :], preferred_element_type=jnp.float32)
           + jnp.dot(hr, ur_ref[:, :], preferred_element_type=jnp.float32)
           + bias_ref[:, :])
    i_g = jax.nn.sigmoid(acc[:, 0:H])
    o_g = jax.nn.sigmoid(acc[:, H:2 * H])
    f_g = jax.nn.sigmoid(acc[:, 2 * H:3 * H])
    u_g = jnp.tanh(acc[:, 3 * H:])
    c_j = i_g * u_g + f_g * (cl + cr)
    h_j = o_g * jnp.tanh(c_j)
    hc = jnp.concatenate([h_j, c_j], axis=1)

    m = mask_ref[:, 0, :, :].reshape(B, 1)
    buf = buf_ref[:, 0, :, :].reshape(B, D)
    row = m * hc + (1.0 - m) * buf
    stack_ref[k + 1] = row

    @pl.when(k == T - 1)
    def _():
        out_ref[:, :] = row[:, :H]


_tc_grid_spec = pltpu.PrefetchScalarGridSpec(
    num_scalar_prefetch=2,
    grid=(T,),
    in_specs=[
        pl.BlockSpec((NW, 1, 16, D), lambda i, s1, s2: (0, i, 0, 0)),
        pl.BlockSpec((NW, 1, 16, 1), lambda i, s1, s2: (0, i, 0, 0)),
        pl.BlockSpec((H, 4 * H), lambda i, s1, s2: (0, 0)),
        pl.BlockSpec((H, 4 * H), lambda i, s1, s2: (0, 0)),
        pl.BlockSpec((1, 4 * H), lambda i, s1, s2: (0, 0)),
    ],
    out_specs=pl.BlockSpec((B, H), lambda i, s1, s2: (0, 0)),
    scratch_shapes=[
        pltpu.VMEM((T + 1, B, D), jnp.float32),
        pltpu.VMEM((B, D), jnp.float32),
        pltpu.VMEM((B, D), jnp.float32),
    ],
)


def kernel(sequence, transitions, Wi, Wf, Wo, Wu, Uil, Uir, Ufl, Ufr,
           Uol, Uor, Uul, Uur, bi, bf, bo, bu):
    del Wi, Wf, Wo, Wu
    trans32 = transitions.astype(jnp.int32)
    trans_prep = trans32.reshape(NW, 16, T).transpose(0, 2, 1).reshape(NW, TB)
    seqflat = sequence.reshape(B * L, D)

    sp1w, sp2w, bufw = _make_sc_call()(trans_prep, seqflat)

    sp1flat = sp1w.reshape(NW * TB)
    sp2flat = sp2w.reshape(NW * TB)
    buf4 = bufw.reshape(NW, T, 16, D)
    maskf = trans_prep.astype(jnp.float32).reshape(NW, T, 16, 1)

    ULcat = jnp.concatenate([Uil.T, Uol.T, Ufl.T, Uul.T], axis=1)
    URcat = jnp.concatenate([Uir.T, Uor.T, Ufr.T, Uur.T], axis=1)
    bcat = jnp.concatenate([bi, bo, bf, bu]).reshape(1, 4 * H)

    return pl.pallas_call(
        _tc_body,
        grid_spec=_tc_grid_spec,
        out_shape=jax.ShapeDtypeStruct((B, H), jnp.float32),
        compiler_params=pltpu.CompilerParams(
            dimension_semantics=("arbitrary",)),
    )(sp1flat, sp2flat, buf4, maskf, ULcat, URcat, bcat)


# speedup vs baseline: 20.7197x; 1.4862x over previous
import functools

import jax
import jax.numpy as jnp
from jax import lax
from jax.experimental import pallas as pl
from jax.experimental.pallas import tpu as pltpu
from jax.experimental.pallas import tpu_sc as plsc

B = 64
L = 255
T = 255
H = 256
D = 2 * H
NW = 4
TB = T * 16
CHUNK = 80
NCHUNK = TB // CHUNK
NSLICE = 8
CPT = 7


def _sc_body(trans_hbm, seq_hbm, sp2_hbm, buf_hbm,
             bq_v, sp2_v, gidx_v, trans_v, rows_a, rows_b,
             sga, sgb, ssa, ssb):
    wid = lax.axis_index("s") * 2 + lax.axis_index("c")
    grp = wid % NW
    slc = wid // NW

    pltpu.sync_copy(trans_hbm.at[grp], trans_v)
    lane = lax.iota(jnp.int32, 16)
    bvec = grp.astype(jnp.int32) * 16 + lane
    zeros = jnp.zeros((16,), jnp.int32)

    def step(t, carry):
        blen, bptr = carry
        i2 = jnp.clip(blen - 2, 0, T - 1)
        q2 = plsc.load_gather(bq_v, [i2 * 16 + lane])
        sp2 = jnp.where(blen >= 2, q2, zeros)
        m = trans_v[pl.ds(t * 16, 16)]
        sp2_v[pl.ds(t * 16, 16)] = sp2
        gidx_v[pl.ds(t * 16, 16)] = bvec * L + jnp.clip(bptr, 0, L - 1)
        nl = jnp.maximum(jnp.where(m == 1, blen - 2, blen), zeros)
        plsc.store_scatter(bq_v, [jnp.clip(nl, 0, T - 1) * 16 + lane],
                           jnp.full((16,), 1, jnp.int32) * (t + 1))
        return (nl + 1, bptr + (1 - m))

    lax.fori_loop(0, T, step, (zeros, zeros))

    @pl.when(slc == 0)
    def _():
        pltpu.sync_copy(sp2_v, sp2_hbm.at[grp])

    bufs = (rows_a, rows_b)
    gsem = (sga, sgb)
    ssem = (ssa, ssb)
    pend = [None, None]
    for j in range(CPT):
        p = j % 2
        c = slc + NSLICE * j
        c = jnp.where(c < NCHUNK, c, c - NCHUNK)
        off = c * CHUNK
        if pend[p] is not None:
            pend[p].wait()
        pltpu.async_copy(
            seq_hbm.at[gidx_v.at[pl.ds(off, CHUNK)]], bufs[p], gsem[p],
        ).wait()
        pend[p] = pltpu.async_copy(
            bufs[p], buf_hbm.at[grp, pl.ds(off, CHUNK), :], ssem[p])
    pend[0].wait()
    pend[1].wait()


@functools.lru_cache(maxsize=1)
def _make_sc_call():
    return functools.partial(
        pl.kernel,
        out_type=(
            jax.ShapeDtypeStruct((NW, TB), jnp.int32),
            jax.ShapeDtypeStruct((NW, TB, D), jnp.float32),
        ),
        mesh=plsc.VectorSubcoreMesh(core_axis_name="c", subcore_axis_name="s"),
        compiler_params=pltpu.CompilerParams(needs_layout_passes=False),
        scratch_types=[
            pltpu.VMEM((TB,), jnp.int32),
            pltpu.VMEM((TB,), jnp.int32),
            pltpu.VMEM((TB,), jnp.int32),
            pltpu.VMEM((TB,), jnp.int32),
            pltpu.VMEM((CHUNK, D), jnp.float32),
            pltpu.VMEM((CHUNK, D), jnp.float32),
            pltpu.SemaphoreType.DMA,
            pltpu.SemaphoreType.DMA,
            pltpu.SemaphoreType.DMA,
            pltpu.SemaphoreType.DMA,
        ],
    )(_sc_body)


def _tc_body(s2_ref, buf_ref, mask_ref, ul_ref, ur_ref, bias_ref,
             out_ref, stack_ref, left_ref):
    k = pl.program_id(0)

    @pl.when(k == 0)
    def _():
        stack_ref[0] = jnp.zeros((B, D), jnp.float32)

    right = stack_ref[k]

    for b in range(B):
        s2 = s2_ref[(b // 16) * TB + k * 16 + (b % 16)]
        left_ref[pl.ds(b, 1), :] = stack_ref[s2, pl.ds(b, 1), :]

    hl = right[:, :H]
    cl = right[:, H:]
    hr = left_ref[:, :H]
    cr = left_ref[:, H:]
    acc = (jnp.dot(hl, ul_ref[:, :], preferred_element_type=jnp.float32)
           + jnp.dot(hr, ur_ref[:, :], preferred_element_type=jnp.float32)
           + bias_ref[:, :])
    i_g = jax.nn.sigmoid(acc[:, 0:H])
    o_g = jax.nn.sigmoid(acc[:, H:2 * H])
    f_g = jax.nn.sigmoid(acc[:, 2 * H:3 * H])
    u_g = jnp.tanh(acc[:, 3 * H:])
    c_j = i_g * u_g + f_g * (cl + cr)
    h_j = o_g * jnp.tanh(c_j)
    hc = jnp.concatenate([h_j, c_j], axis=1)

    m = mask_ref[:, 0, :, :].reshape(B, 1)
    buf = buf_ref[:, :, :].reshape(B, D)
    row = m * hc + (1.0 - m) * buf
    stack_ref[k + 1] = row

    @pl.when(k == T - 1)
    def _():
        out_ref[:, :] = row[:, :H]


_tc_grid_spec = pltpu.PrefetchScalarGridSpec(
    num_scalar_prefetch=1,
    grid=(T,),
    in_specs=[
        pl.BlockSpec((NW, 16, D), lambda i, s2: (0, i, 0)),
        pl.BlockSpec((NW, 1, 16, 1), lambda i, s2: (0, i, 0, 0)),
        pl.BlockSpec((H, 4 * H), lambda i, s2: (0, 0)),
        pl.BlockSpec((H, 4 * H), lambda i, s2: (0, 0)),
        pl.BlockSpec((1, 4 * H), lambda i, s2: (0, 0)),
    ],
    out_specs=pl.BlockSpec((B, H), lambda i, s2: (0, 0)),
    scratch_shapes=[
        pltpu.VMEM((T + 1, B, D), jnp.float32),
        pltpu.VMEM((B, D), jnp.float32),
    ],
)


def kernel(sequence, transitions, Wi, Wf, Wo, Wu, Uil, Uir, Ufl, Ufr,
           Uol, Uor, Uul, Uur, bi, bf, bo, bu):
    del Wi, Wf, Wo, Wu
    trans32 = transitions.astype(jnp.int32)
    trans_prep = trans32.reshape(NW, 16, T).transpose(0, 2, 1).reshape(NW, TB)
    seqflat = sequence.reshape(B * L, D)

    sp2w, bufw = _make_sc_call()(trans_prep, seqflat)

    sp2flat = sp2w.reshape(NW * TB)
    maskf = trans_prep.astype(jnp.float32).reshape(NW, T, 16, 1)

    ULcat = jnp.concatenate([Uil.T, Uol.T, Ufl.T, Uul.T], axis=1)
    URcat = jnp.concatenate([Uir.T, Uor.T, Ufr.T, Uur.T], axis=1)
    bcat = jnp.concatenate([bi, bo, bf, bu]).reshape(1, 4 * H)

    return pl.pallas_call(
        _tc_body,
        grid_spec=_tc_grid_spec,
        out_shape=jax.ShapeDtypeStruct((B, H), jnp.float32),
        compiler_params=pltpu.CompilerParams(
            dimension_semantics=("arbitrary",)),
    )(sp2flat, bufw, maskf, ULcat, URcat, bcat)


# speedup vs baseline: 20.8920x; 1.0083x over previous
import functools

import jax
import jax.numpy as jnp
from jax import lax
from jax.experimental import pallas as pl
from jax.experimental.pallas import tpu as pltpu
from jax.experimental.pallas import tpu_sc as plsc

B = 64
L = 255
T = 255
H = 256
D = 2 * H
NW = 4
TB = T * 16
CHUNK = 80
NCHUNK = TB // CHUNK
NSLICE = 8
CPT = 7


def _sc_body(trans_hbm, seq_hbm, sp2_hbm, bufw_hbm,
             bq_v, sp2_v, gidx_v, trans_v,
             rows0_v, rows1_v, sem0, sem1):
    wid = lax.axis_index("s") * 2 + lax.axis_index("c")
    grp = wid % NW
    slc = wid // NW

    pltpu.sync_copy(trans_hbm.at[grp], trans_v)
    lane = lax.iota(jnp.int32, 16)
    zeros = jnp.zeros((16,), jnp.int32)
    base = (grp * 16 + lane) * L

    def step(t, carry):
        blen, bptr = carry
        i2 = jnp.clip(blen - 2, 0, T - 1)
        q2 = plsc.load_gather(bq_v, [i2 * 16 + lane])
        sp2 = jnp.where(blen >= 2, q2, zeros)
        m = trans_v[pl.ds(t * 16, 16)]
        sp2_v[pl.ds(t * 16, 16)] = sp2
        gidx_v[pl.ds(t * 16, 16)] = base + jnp.clip(bptr, 0, L - 1)
        nl = jnp.maximum(jnp.where(m == 1, blen - 2, blen), zeros)
        plsc.store_scatter(bq_v, [jnp.clip(nl, 0, T - 1) * 16 + lane],
                           jnp.full((16,), 1, jnp.int32) * (t + 1))
        return (nl + 1, bptr + (1 - m))

    lax.fori_loop(0, T, step, (zeros, zeros))

    @pl.when(slc == 0)
    def _():
        pltpu.sync_copy(sp2_v, sp2_hbm.at[grp])

    bufs = ((rows0_v, sem0), (rows1_v, sem1))

    def start(c):
        cid = c * NSLICE + slc
        rows_v, sem = bufs[c % 2]

        @pl.when(cid < NCHUNK)
        def _():
            pltpu.async_copy(
                seq_hbm.at[gidx_v.at[pl.ds(cid * CHUNK, CHUNK)]], rows_v, sem)

    def drain(c):
        cid = c * NSLICE + slc
        rows_v, sem = bufs[c % 2]

        @pl.when(cid < NCHUNK)
        def _():
            pltpu.make_async_copy(
                seq_hbm.at[gidx_v.at[pl.ds(cid * CHUNK, CHUNK)]],
                rows_v, sem).wait()
            pltpu.sync_copy(rows_v, bufw_hbm.at[grp, pl.ds(cid * CHUNK, CHUNK)])

    start(0)
    for c in range(1, CPT):
        start(c)
        drain(c - 1)
    drain(CPT - 1)


@functools.lru_cache(maxsize=1)
def _make_sc_call():
    return functools.partial(
        pl.kernel,
        out_type=(
            jax.ShapeDtypeStruct((NW, TB), jnp.int32),
            jax.ShapeDtypeStruct((NW, TB, D), jnp.float32),
        ),
        mesh=plsc.VectorSubcoreMesh(core_axis_name="c", subcore_axis_name="s"),
        compiler_params=pltpu.CompilerParams(needs_layout_passes=False),
        scratch_types=[
            pltpu.VMEM((TB,), jnp.int32),
            pltpu.VMEM((TB,), jnp.int32),
            pltpu.VMEM((TB,), jnp.int32),
            pltpu.VMEM((TB,), jnp.int32),
            pltpu.VMEM((CHUNK, D), jnp.float32),
            pltpu.VMEM((CHUNK, D), jnp.float32),
            pltpu.SemaphoreType.DMA,
            pltpu.SemaphoreType.DMA,
        ],
    )(_sc_body)


def _tc_body(s2_ref, bufw_ref, mask_ref, ul_ref, ur_ref, bias_ref,
             out_ref, stack_ref, left_ref):
    k = pl.program_id(0)

    @pl.when(k == 0)
    def _():
        stack_ref[0] = jnp.zeros((B, D), jnp.float32)

    right = stack_ref[k]

    for b in range(B):
        flat = (b // 16) * TB + k * 16 + (b % 16)
        s2 = s2_ref[flat]
        left_ref[pl.ds(b, 1), :] = stack_ref[s2, pl.ds(b, 1), :]

    hl = right[:, :H]
    cl = right[:, H:]
    hr = left_ref[:, :H]
    cr = left_ref[:, H:]
    acc = (jnp.dot(hl, ul_ref[:, :], preferred_element_type=jnp.float32)
           + jnp.dot(hr, ur_ref[:, :], preferred_element_type=jnp.float32)
           + bias_ref[:, :])
    i_g = jax.nn.sigmoid(acc[:, 0:H])
    o_g = jax.nn.sigmoid(acc[:, H:2 * H])
    f_g = jax.nn.sigmoid(acc[:, 2 * H:3 * H])
    u_g = jnp.tanh(acc[:, 3 * H:])
    c_j = i_g * u_g + f_g * (cl + cr)
    h_j = o_g * jnp.tanh(c_j)
    hc = jnp.concatenate([h_j, c_j], axis=1)

    m = mask_ref[:, 0, :, :].reshape(B, 1)
    bufv = bufw_ref[:, 0, :, :].reshape(B, D)
    row = m * hc + (1.0 - m) * bufv
    stack_ref[k + 1] = row

    @pl.when(k == T - 1)
    def _():
        out_ref[:, :] = row[:, :H]


_tc_grid_spec = pltpu.PrefetchScalarGridSpec(
    num_scalar_prefetch=1,
    grid=(T,),
    in_specs=[
        pl.BlockSpec((NW, 1, 16, D), lambda i, s2: (0, i, 0, 0)),
        pl.BlockSpec((NW, 1, 16, 1), lambda i, s2: (0, i, 0, 0)),
        pl.BlockSpec((H, 4 * H), lambda i, s2: (0, 0)),
        pl.BlockSpec((H, 4 * H), lambda i, s2: (0, 0)),
        pl.BlockSpec((1, 4 * H), lambda i, s2: (0, 0)),
    ],
    out_specs=pl.BlockSpec((B, H), lambda i, s2: (0, 0)),
    scratch_shapes=[
        pltpu.VMEM((T + 1, B, D), jnp.float32),
        pltpu.VMEM((B, D), jnp.float32),
    ],
)


def kernel(sequence, transitions, Wi, Wf, Wo, Wu, Uil, Uir, Ufl, Ufr,
           Uol, Uor, Uul, Uur, bi, bf, bo, bu):
    del Wi, Wf, Wo, Wu
    trans32 = transitions.astype(jnp.int32)
    trans_prep = trans32.reshape(NW, 16, T).transpose(0, 2, 1).reshape(NW, TB)
    seqflat = sequence.reshape(B * L, D)

    sp2w, bufw = _make_sc_call()(trans_prep, seqflat)

    sp2flat = sp2w.reshape(NW * TB)
    bufw4 = bufw.reshape(NW, T, 16, D)
    maskf = trans_prep.astype(jnp.float32).reshape(NW, T, 16, 1)

    ULcat = jnp.concatenate([Uil.T, Uol.T, Ufl.T, Uul.T], axis=1)
    URcat = jnp.concatenate([Uir.T, Uor.T, Ufr.T, Uur.T], axis=1)
    bcat = jnp.concatenate([bi, bo, bf, bu]).reshape(1, 4 * H)

    return pl.pallas_call(
        _tc_body,
        grid_spec=_tc_grid_spec,
        out_shape=jax.ShapeDtypeStruct((B, H), jnp.float32),
        compiler_params=pltpu.CompilerParams(
            dimension_semantics=("arbitrary",),
            vmem_limit_bytes=110 * 1024 * 1024),
    )(sp2flat, bufw4, maskf, ULcat, URcat, bcat)


# speedup vs baseline: 20.9753x; 1.0040x over previous
import functools

import jax
import jax.numpy as jnp
from jax import lax
from jax.experimental import pallas as pl
from jax.experimental.pallas import tpu as pltpu
from jax.experimental.pallas import tpu_sc as plsc

B = 64
L = 255
T = 255
H = 256
D = 2 * H
NW = 4
TB = T * 16
CHUNK = 80
NCHUNK = TB // CHUNK
NSLICE = 8
CPT = 7


def _sc_body(trans_hbm, seq_hbm, sp2_hbm, bufw_hbm,
             bq_v, sp2_v, gidx_v, trans_v,
             rows0_v, rows1_v, sem0, sem1):
    wid = lax.axis_index("s") * 2 + lax.axis_index("c")
    grp = wid % NW
    slc = wid // NW

    pltpu.sync_copy(trans_hbm.at[grp], trans_v)
    lane = lax.iota(jnp.int32, 16)
    zeros = jnp.zeros((16,), jnp.int32)
    base = (grp * 16 + lane) * L

    def step(t, carry):
        blen, bptr = carry
        i2 = jnp.clip(blen - 2, 0, T - 1)
        q2 = plsc.load_gather(bq_v, [i2 * 16 + lane])
        sp2 = jnp.where(blen >= 2, q2, zeros)
        m = trans_v[pl.ds(t * 16, 16)]
        sp2_v[pl.ds(t * 16, 16)] = sp2
        gidx_v[pl.ds(t * 16, 16)] = base + jnp.clip(bptr, 0, L - 1)
        nl = jnp.maximum(jnp.where(m == 1, blen - 2, blen), zeros)
        plsc.store_scatter(bq_v, [jnp.clip(nl, 0, T - 1) * 16 + lane],
                           jnp.full((16,), 1, jnp.int32) * (t + 1))
        return (nl + 1, bptr + (1 - m))

    lax.fori_loop(0, T, step, (zeros, zeros))

    @pl.when(slc == 0)
    def _():
        pltpu.sync_copy(sp2_v, sp2_hbm.at[grp])

    bufs = ((rows0_v, sem0), (rows1_v, sem1))

    def start(c):
        cid = c * NSLICE + slc
        rows_v, sem = bufs[c % 2]

        @pl.when(cid < NCHUNK)
        def _():
            pltpu.async_copy(
                seq_hbm.at[gidx_v.at[pl.ds(cid * CHUNK, CHUNK)]], rows_v, sem)

    def drain(c):
        cid = c * NSLICE + slc
        rows_v, sem = bufs[c % 2]

        @pl.when(cid < NCHUNK)
        def _():
            pltpu.make_async_copy(
                seq_hbm.at[gidx_v.at[pl.ds(cid * CHUNK, CHUNK)]],
                rows_v, sem).wait()
            pltpu.sync_copy(rows_v, bufw_hbm.at[grp, pl.ds(cid * CHUNK, CHUNK)])

    start(0)
    for c in range(1, CPT):
        start(c)
        drain(c - 1)
    drain(CPT - 1)


@functools.lru_cache(maxsize=1)
def _make_sc_call():
    return functools.partial(
        pl.kernel,
        out_type=(
            jax.ShapeDtypeStruct((NW, TB), jnp.int32),
            jax.ShapeDtypeStruct((NW, TB, D), jnp.float32),
        ),
        mesh=plsc.VectorSubcoreMesh(core_axis_name="c", subcore_axis_name="s"),
        compiler_params=pltpu.CompilerParams(needs_layout_passes=False),
        scratch_types=[
            pltpu.VMEM((TB,), jnp.int32),
            pltpu.VMEM((TB,), jnp.int32),
            pltpu.VMEM((TB,), jnp.int32),
            pltpu.VMEM((TB,), jnp.int32),
            pltpu.VMEM((CHUNK, D), jnp.float32),
            pltpu.VMEM((CHUNK, D), jnp.float32),
            pltpu.SemaphoreType.DMA,
            pltpu.SemaphoreType.DMA,
        ],
    )(_sc_body)


def _tc_body(s2_ref, bufw_ref, mask_ref, ul_ref, ur_ref, bias_ref,
             out_ref, stack_ref):
    k = pl.program_id(0)

    @pl.when(k == 0)
    def _():
        stack_ref[0] = jnp.zeros((B, D), jnp.float32)

    right = stack_ref[k]

    rows = []
    for b in range(B):
        flat = (b // 16) * TB + k * 16 + (b % 16)
        s2 = s2_ref[flat]
        rows.append(stack_ref[s2, pl.ds(b, 1), :])
    left = jnp.concatenate(rows, axis=0)

    hl = right[:, :H]
    cl = right[:, H:]
    hr = left[:, :H]
    cr = left[:, H:]
    acc = (jnp.dot(hl, ul_ref[:, :], preferred_element_type=jnp.float32)
           + jnp.dot(hr, ur_ref[:, :], preferred_element_type=jnp.float32)
           + bias_ref[:, :])
    i_g = jax.nn.sigmoid(acc[:, 0:H])
    o_g = jax.nn.sigmoid(acc[:, H:2 * H])
    f_g = jax.nn.sigmoid(acc[:, 2 * H:3 * H])
    u_g = jnp.tanh(acc[:, 3 * H:])
    c_j = i_g * u_g + f_g * (cl + cr)
    h_j = o_g * jnp.tanh(c_j)
    hc = jnp.concatenate([h_j, c_j], axis=1)

    m = mask_ref[:, 0, :, :].reshape(B, 1)
    bufv = bufw_ref[:, 0, :, :].reshape(B, D)
    row = m * hc + (1.0 - m) * bufv
    stack_ref[k + 1] = row

    @pl.when(k == T - 1)
    def _():
        out_ref[:, :] = row[:, :H]


_tc_grid_spec = pltpu.PrefetchScalarGridSpec(
    num_scalar_prefetch=1,
    grid=(T,),
    in_specs=[
        pl.BlockSpec((NW, 1, 16, D), lambda i, s2: (0, i, 0, 0)),
        pl.BlockSpec((NW, 1, 16, 1), lambda i, s2: (0, i, 0, 0)),
        pl.BlockSpec((H, 4 * H), lambda i, s2: (0, 0)),
        pl.BlockSpec((H, 4 * H), lambda i, s2: (0, 0)),
        pl.BlockSpec((1, 4 * H), lambda i, s2: (0, 0)),
    ],
    out_specs=pl.BlockSpec((B, H), lambda i, s2: (0, 0)),
    scratch_shapes=[
        pltpu.VMEM((T + 1, B, D), jnp.float32),
    ],
)


def kernel(sequence, transitions, Wi, Wf, Wo, Wu, Uil, Uir, Ufl, Ufr,
           Uol, Uor, Uul, Uur, bi, bf, bo, bu):
    del Wi, Wf, Wo, Wu
    trans32 = transitions.astype(jnp.int32)
    trans_prep = trans32.reshape(NW, 16, T).transpose(0, 2, 1).reshape(NW, TB)
    seqflat = sequence.reshape(B * L, D)

    sp2w, bufw = _make_sc_call()(trans_prep, seqflat)

    sp2flat = sp2w.reshape(NW * TB)
    bufw4 = bufw.reshape(NW, T, 16, D)
    maskf = trans_prep.astype(jnp.float32).reshape(NW, T, 16, 1)

    ULcat = jnp.concatenate([Uil.T, Uol.T, Ufl.T, Uul.T], axis=1)
    URcat = jnp.concatenate([Uir.T, Uor.T, Ufr.T, Uur.T], axis=1)
    bcat = jnp.concatenate([bi, bo, bf, bu]).reshape(1, 4 * H)

    return pl.pallas_call(
        _tc_body,
        grid_spec=_tc_grid_spec,
        out_shape=jax.ShapeDtypeStruct((B, H), jnp.float32),
        compiler_params=pltpu.CompilerParams(
            dimension_semantics=("arbitrary",),
            vmem_limit_bytes=110 * 1024 * 1024),
    )(sp2flat, bufw4, maskf, ULcat, URcat, bcat)
